# Initial kernel scaffold; baseline (speedup 1.0000x reference)
import functools

import jax
import jax.numpy as jnp
from jax import lax
from jax.experimental import pallas as pl
from jax.experimental.pallas import tpu as pltpu
from jax.experimental.pallas import tpu_sc as plsc

N = 10000
E = 320000
G = 16
H = 256

NT = 16
K = 80
EP = E // NT
NCHUNK = EP // K
RPT = N // NT

BN = 1000
NB = N // BN


def _sc_aggregate(h2, src3, dst3):
    fh = h2.shape[2]
    mesh = plsc.VectorSubcoreMesh(core_axis_name="c", subcore_axis_name="s")

    @functools.partial(
        pl.kernel,
        mesh=mesh,
        out_type=jax.ShapeDtypeStruct((2, N, fh), jnp.float32),
        scratch_types=[
            pltpu.VMEM((NCHUNK, K), jnp.int32),
            pltpu.VMEM((NCHUNK, K), jnp.int32),
            pltpu.VMEM((2, K, fh), jnp.float32),
            pltpu.VMEM_SHARED((N, fh), jnp.float32),
            pltpu.SemaphoreType.DMA,
            pltpu.SemaphoreType.DMA,
        ],
    )
    def agg(h_hbm, src_hbm, dst_hbm, out_hbm, src_v, dst_v, rows_v, slab, sem0, sem1):
        c = lax.axis_index("c")
        s = lax.axis_index("s")
        sems = (sem0, sem1)

        pltpu.sync_copy(h_hbm.at[c, pl.ds(s * RPT, RPT)], slab.at[pl.ds(s * RPT, RPT)])
        pltpu.sync_copy(src_hbm.at[s], src_v)
        pltpu.sync_copy(dst_hbm.at[s], dst_v)
        plsc.subcore_barrier()

        def gather(chunk, buf):
            return pltpu.make_async_copy(
                h_hbm.at[c].at[src_v.at[chunk]], rows_v.at[buf], sems[buf])

        gather(0, 0).start()

        def body(i, _):
            g = i * 2
            for b in range(2):
                cur = g + b
                nxt = cur + 1

                @pl.when(nxt < NCHUNK)
                def _():
                    gather(nxt, 1 - b).start()

                gather(cur, b).wait()
                pltpu.sync_copy(rows_v.at[b], slab.at[dst_v.at[cur]], add=True)
            return ()

        lax.fori_loop(0, NCHUNK // 2, body, (), unroll=False)

        plsc.subcore_barrier()
        pltpu.sync_copy(slab.at[pl.ds(s * RPT, RPT)], out_hbm.at[c, pl.ds(s * RPT, RPT)])

    return agg(h2, src3, dst3)


def _tc_layer(h2, W1s, b1, W2, b2, gamma, beta):
    fh = h2.shape[2]
    oh = H // 2

    def body(h_ref, W1_ref, b1_ref, W2_ref, b2_ref, g_ref, be_ref, o_ref,
             u_scr, stat_scr):
        p = pl.program_id(0)
        j = pl.program_id(1)

        @pl.when(p == 0)
        def _():
            t = jnp.dot(h_ref[0], W1_ref[0], preferred_element_type=jnp.float32)
            t += jnp.dot(h_ref[1], W1_ref[1], preferred_element_type=jnp.float32)
            t = jnp.maximum(t + b1_ref[...], 0.0)
            u = jnp.dot(t, W2_ref[...], preferred_element_type=jnp.float32) + b2_ref[...]
            u_scr[pl.ds(j * BN, BN)] = u

            @pl.when(j == 0)
            def _():
                stat_scr[...] = jnp.zeros_like(stat_scr)

            stat_scr[0:1] += jnp.sum(u, axis=0, keepdims=True)
            stat_scr[1:2] += jnp.sum(u * u, axis=0, keepdims=True)

        @pl.when(p == 1)
        def _():
            mu = stat_scr[0:1] / N
            var = stat_scr[1:2] / N - mu * mu
            u = u_scr[pl.ds(j * BN, BN)]
            y = g_ref[...] * (u - mu) / jnp.sqrt(var + 1e-5) + be_ref[...]
            y = jnp.maximum(y, 0.0)
            o_ref[0] = y[:, :oh]
            o_ref[1] = y[:, oh:]

    return pl.pallas_call(
        body,
        grid=(2, NB),
        in_specs=[
            pl.BlockSpec((2, BN, fh), lambda p, j: (0, j, 0)),
            pl.BlockSpec((2, fh, H), lambda p, j: (0, 0, 0)),
            pl.BlockSpec((1, H), lambda p, j: (0, 0)),
            pl.BlockSpec((H, H), lambda p, j: (0, 0)),
            pl.BlockSpec((1, H), lambda p, j: (0, 0)),
            pl.BlockSpec((1, H), lambda p, j: (0, 0)),
            pl.BlockSpec((1, H), lambda p, j: (0, 0)),
        ],
        out_specs=pl.BlockSpec((2, BN, oh), lambda p, j: (0, j, 0)),
        out_shape=jax.ShapeDtypeStruct((2, N, oh), jnp.float32),
        scratch_shapes=[
            pltpu.VMEM((N, H), jnp.float32),
            pltpu.VMEM((2, H), jnp.float32),
        ],
    )(h2, W1s, b1, W2, b2, gamma, beta)


def _readout(h2, gids3, Wf1, bf1, Wf2, bf2, g4, be4):
    F1 = Wf1.shape[1]
    DOUT = Wf2.shape[1]
    oh = H // 2

    def body(h_ref, gid_ref, Wf1_ref, bf1_ref, Wf2_ref, bf2_ref, g4_ref,
             be4_ref, o_ref, hg_scr):
        j = pl.program_id(0)

        @pl.when(j == 0)
        def _():
            hg_scr[...] = jnp.zeros_like(hg_scr)

        gid = gid_ref[0, 0, :]
        onehot = (gid[None, :] == lax.broadcasted_iota(jnp.int32, (G, BN), 0)
                  ).astype(jnp.float32)
        hg_scr[:, :oh] += jnp.dot(onehot, h_ref[0], preferred_element_type=jnp.float32)
        hg_scr[:, oh:] += jnp.dot(onehot, h_ref[1], preferred_element_type=jnp.float32)

        @pl.when(j == NB - 1)
        def _():
            hg = hg_scr[...]
            t = jnp.dot(hg, Wf1_ref[...], preferred_element_type=jnp.float32) + bf1_ref[...]
            mu = jnp.mean(t, axis=0, keepdims=True)
            var = jnp.mean(t * t, axis=0, keepdims=True) - mu * mu
            t = g4_ref[...] * (t - mu) / jnp.sqrt(var + 1e-5) + be4_ref[...]
            t = jnp.maximum(t, 0.0) + jnp.log1p(jnp.exp(-jnp.abs(t)))
            o = jnp.dot(t, Wf2_ref[...], preferred_element_type=jnp.float32) + bf2_ref[...]
            nrm = jnp.maximum(jnp.sqrt(jnp.sum(o * o, axis=1, keepdims=True)), 1e-12)
            o_ref[...] = o / nrm

    return pl.pallas_call(
        body,
        grid=(NB,),
        in_specs=[
            pl.BlockSpec((2, BN, oh), lambda j: (0, j, 0)),
            pl.BlockSpec((1, 1, BN), lambda j: (j, 0, 0)),
            pl.BlockSpec((H, F1), lambda j: (0, 0)),
            pl.BlockSpec((1, F1), lambda j: (0, 0)),
            pl.BlockSpec((F1, DOUT), lambda j: (0, 0)),
            pl.BlockSpec((1, DOUT), lambda j: (0, 0)),
            pl.BlockSpec((1, F1), lambda j: (0, 0)),
            pl.BlockSpec((1, F1), lambda j: (0, 0)),
        ],
        out_specs=pl.BlockSpec((G, DOUT), lambda j: (0, 0)),
        out_shape=jax.ShapeDtypeStruct((G, DOUT), jnp.float32),
        scratch_shapes=[pltpu.VMEM((G, H), jnp.float32)],
    )(h2, gids3, Wf1, bf1, Wf2, bf2, g4, be4)


def kernel(x, edge_index, graph_ids, params):
    src3 = edge_index[0].reshape(NT, NCHUNK, K)
    dst3 = edge_index[1].reshape(NT, NCHUNK, K)
    gids3 = graph_ids.reshape(NB, 1, BN)

    D = x.shape[1]
    h2 = x.reshape(N, 2, D // 2).transpose(1, 0, 2)
    for i in (1, 2, 3):
        p = params['gc%d' % i]
        bn = params['bn%d' % i]
        fin = 2 * h2.shape[2]
        a2 = _sc_aggregate(h2, src3, dst3)
        h2 = _tc_layer(
            a2,
            p['W1'].reshape(2, fin // 2, H),
            p['b1'].reshape(1, H),
            p['W2'],
            p['b2'].reshape(1, H),
            bn['gamma'].reshape(1, H),
            bn['beta'].reshape(1, H),
        )
    return _readout(
        h2, gids3,
        params['fc1']['W'], params['fc1']['b'].reshape(1, -1),
        params['fc2']['W'], params['fc2']['b'].reshape(1, -1),
        params['bn4']['gamma'].reshape(1, -1), params['bn4']['beta'].reshape(1, -1),
    )


# speedup vs baseline: 7.7583x; 7.7583x over previous
import functools

import jax
import jax.numpy as jnp
from jax import lax
from jax.experimental import pallas as pl
from jax.experimental.pallas import tpu as pltpu
from jax.experimental.pallas import tpu_sc as plsc

N = 10000
E = 320000
G = 16
H = 256

NT = 16
K = 80
TB = 25
EP = E // NT
NCHUNK = EP // K
RPT = 624
RTAIL = N - NT * RPT

BN = 1000
NB = N // BN


def _slab_copy(src_ref, dst_ref, s):
    pltpu.sync_copy(src_ref.at[pl.ds(s * RPT, RPT)], dst_ref.at[pl.ds(s * RPT, RPT)])

    @pl.when(s == NT - 1)
    def _():
        pltpu.sync_copy(src_ref.at[pl.ds(NT * RPT, RTAIL)],
                        dst_ref.at[pl.ds(NT * RPT, RTAIL)])


def _edge_loop(nblk, load_idx_block, gather_src, src_v, dst_v, rows_v, slab,
               sems):
    def gather(chunk, buf):
        return pltpu.make_async_copy(
            gather_src.at[src_v.at[chunk]], rows_v.at[buf], sems[buf])

    def blk_body(blk, _):
        load_idx_block(blk)
        gather(0, 0).start()
        for i in range(TB):
            if i + 1 < TB:
                gather(i + 1, (i + 1) % 2).start()
            gather(i, i % 2).wait()
            pltpu.sync_copy(rows_v.at[i % 2], slab.at[dst_v.at[i]], add=True)
        return ()

    lax.fori_loop(0, nblk, blk_body, (), unroll=False)


def _sc_aggregate_edges(h, zeros, src5, dst5):
    fh = h.shape[1]
    nblk = src5.shape[2]
    mesh = plsc.VectorSubcoreMesh(
        core_axis_name="c", subcore_axis_name="s", num_cores=2, num_subcores=NT)

    @functools.partial(
        pl.kernel,
        mesh=mesh,
        out_type=jax.ShapeDtypeStruct((2, N, fh), jnp.float32),
        scratch_types=[
            pltpu.VMEM((TB, K), jnp.int32),
            pltpu.VMEM((TB, K), jnp.int32),
            pltpu.VMEM((2, K, fh), jnp.float32),
            pltpu.VMEM_SHARED((N, fh), jnp.float32),
            pltpu.SemaphoreType.DMA,
            pltpu.SemaphoreType.DMA,
        ],
    )
    def agg(h_hbm, z_hbm, src_hbm, dst_hbm, out_hbm, src_v, dst_v, rows_v,
            slab, sem0, sem1):
        c = lax.axis_index("c")
        s = lax.axis_index("s")

        @pl.when(c == 0)
        def _():
            _slab_copy(h_hbm, slab, s)

        @pl.when(c == 1)
        def _():
            _slab_copy(z_hbm, slab, s)

        plsc.subcore_barrier()

        def load_idx_block(blk):
            pltpu.sync_copy(src_hbm.at[c, s, blk], src_v)
            pltpu.sync_copy(dst_hbm.at[c, s, blk], dst_v)

        _edge_loop(nblk, load_idx_block, h_hbm, src_v, dst_v, rows_v, slab,
                   (sem0, sem1))

        plsc.subcore_barrier()
        _slab_copy(slab, out_hbm.at[c], s)

    return agg(h, zeros, src5, dst5)


def _sc_aggregate(h2, src4, dst4):
    fh = h2.shape[2]
    nblk = src4.shape[1]
    mesh = plsc.VectorSubcoreMesh(
        core_axis_name="c", subcore_axis_name="s", num_cores=2, num_subcores=NT)

    @functools.partial(
        pl.kernel,
        mesh=mesh,
        out_type=jax.ShapeDtypeStruct((2, N, fh), jnp.float32),
        scratch_types=[
            pltpu.VMEM((TB, K), jnp.int32),
            pltpu.VMEM((TB, K), jnp.int32),
            pltpu.VMEM((2, K, fh), jnp.float32),
            pltpu.VMEM_SHARED((N, fh), jnp.float32),
            pltpu.SemaphoreType.DMA,
            pltpu.SemaphoreType.DMA,
        ],
    )
    def agg(h_hbm, src_hbm, dst_hbm, out_hbm, src_v, dst_v, rows_v, slab,
            sem0, sem1):
        c = lax.axis_index("c")
        s = lax.axis_index("s")

        _slab_copy(h_hbm.at[c], slab, s)
        plsc.subcore_barrier()

        def load_idx_block(blk):
            pltpu.sync_copy(src_hbm.at[s, blk], src_v)
            pltpu.sync_copy(dst_hbm.at[s, blk], dst_v)

        _edge_loop(nblk, load_idx_block, h_hbm.at[c], src_v, dst_v, rows_v,
                   slab, (sem0, sem1))

        plsc.subcore_barrier()
        _slab_copy(slab, out_hbm.at[c], s)

    return agg(h2, src4, dst4)


def _tc_layer(h2, W1s, b1, W2, b2, gamma, beta, sum_parts=False):
    fh = h2.shape[2]
    oh = H // 2

    def mlp_body(h_ref, W1_ref, b1_ref, W2_ref, b2_ref, u_ref, stat_ref,
                 stat_scr):
        j = pl.program_id(0)
        if sum_parts:
            t = jnp.dot(h_ref[0] + h_ref[1], W1_ref[0],
                        preferred_element_type=jnp.float32)
        else:
            t = jnp.dot(h_ref[0], W1_ref[0], preferred_element_type=jnp.float32)
            t += jnp.dot(h_ref[1], W1_ref[1], preferred_element_type=jnp.float32)
        t = jnp.maximum(t + b1_ref[...], 0.0)
        u = jnp.dot(t, W2_ref[...], preferred_element_type=jnp.float32) + b2_ref[...]
        u_ref[...] = u

        bsum = jnp.sum(u, axis=0, keepdims=True)
        mu_b = bsum / BN
        msum = jnp.sum((u - mu_b) ** 2, axis=0, keepdims=True)
        stat_scr[pl.ds(j, 1)] = bsum
        stat_scr[pl.ds(NB + j, 1)] = msum
        stat_ref[...] = stat_scr[...]

    u, stats = pl.pallas_call(
        mlp_body,
        grid=(NB,),
        in_specs=[
            pl.BlockSpec((2, BN, fh), lambda j: (0, j, 0)),
            pl.BlockSpec(W1s.shape, lambda j: (0, 0, 0)),
            pl.BlockSpec((1, H), lambda j: (0, 0)),
            pl.BlockSpec((H, H), lambda j: (0, 0)),
            pl.BlockSpec((1, H), lambda j: (0, 0)),
        ],
        out_specs=[
            pl.BlockSpec((BN, H), lambda j: (j, 0)),
            pl.BlockSpec((2 * NB, H), lambda j: (0, 0)),
        ],
        out_shape=[
            jax.ShapeDtypeStruct((N, H), jnp.float32),
            jax.ShapeDtypeStruct((2 * NB, H), jnp.float32),
        ],
        scratch_shapes=[pltpu.VMEM((2 * NB, H), jnp.float32)],
    )(h2, W1s, b1, W2, b2)

    def bn_body(u_ref, stat_ref, g_ref, be_ref, o_ref):
        bsums = stat_ref[0:NB]
        msums = stat_ref[NB:2 * NB]
        mu = jnp.sum(bsums, axis=0, keepdims=True) / N
        mu_b = bsums / BN
        var = (jnp.sum(msums, axis=0, keepdims=True)
               + BN * jnp.sum((mu_b - mu) ** 2, axis=0, keepdims=True)) / N
        y = g_ref[...] * (u_ref[...] - mu) / jnp.sqrt(var + 1e-5) + be_ref[...]
        y = jnp.maximum(y, 0.0)
        o_ref[0] = y[:, :oh]
        o_ref[1] = y[:, oh:]

    return pl.pallas_call(
        bn_body,
        grid=(NB,),
        in_specs=[
            pl.BlockSpec((BN, H), lambda j: (j, 0)),
            pl.BlockSpec((2 * NB, H), lambda j: (0, 0)),
            pl.BlockSpec((1, H), lambda j: (0, 0)),
            pl.BlockSpec((1, H), lambda j: (0, 0)),
        ],
        out_specs=pl.BlockSpec((2, BN, oh), lambda j: (0, j, 0)),
        out_shape=jax.ShapeDtypeStruct((2, N, oh), jnp.float32),
    )(u, stats, gamma, beta)


def _readout(h2, gids3, Wf1, bf1, Wf2, bf2, g4, be4):
    F1 = Wf1.shape[1]
    DOUT = Wf2.shape[1]
    oh = H // 2

    def body(h_ref, gid_ref, Wf1_ref, bf1_ref, Wf2_ref, bf2_ref, g4_ref,
             be4_ref, o_ref, hg_scr):
        j = pl.program_id(0)

        @pl.when(j == 0)
        def _():
            hg_scr[...] = jnp.zeros_like(hg_scr)

        gid = gid_ref[0, 0, :]
        onehot = (gid[None, :] == lax.broadcasted_iota(jnp.int32, (G, BN), 0)
                  ).astype(jnp.float32)
        hg_scr[:, :oh] += jnp.dot(onehot, h_ref[0], preferred_element_type=jnp.float32)
        hg_scr[:, oh:] += jnp.dot(onehot, h_ref[1], preferred_element_type=jnp.float32)

        @pl.when(j == NB - 1)
        def _():
            hg = hg_scr[...]
            t = jnp.dot(hg, Wf1_ref[...], preferred_element_type=jnp.float32) + bf1_ref[...]
            mu = jnp.mean(t, axis=0, keepdims=True)
            var = jnp.mean((t - mu) ** 2, axis=0, keepdims=True)
            t = g4_ref[...] * (t - mu) / jnp.sqrt(var + 1e-5) + be4_ref[...]
            t = jnp.maximum(t, 0.0) + jnp.log1p(jnp.exp(-jnp.abs(t)))
            o = jnp.dot(t, Wf2_ref[...], preferred_element_type=jnp.float32) + bf2_ref[...]
            nrm = jnp.maximum(jnp.sqrt(jnp.sum(o * o, axis=1, keepdims=True)), 1e-12)
            o_ref[...] = o / nrm

    return pl.pallas_call(
        body,
        grid=(NB,),
        in_specs=[
            pl.BlockSpec((2, BN, oh), lambda j: (0, j, 0)),
            pl.BlockSpec((1, 1, BN), lambda j: (j, 0, 0)),
            pl.BlockSpec((H, F1), lambda j: (0, 0)),
            pl.BlockSpec((1, F1), lambda j: (0, 0)),
            pl.BlockSpec((F1, DOUT), lambda j: (0, 0)),
            pl.BlockSpec((1, DOUT), lambda j: (0, 0)),
            pl.BlockSpec((1, F1), lambda j: (0, 0)),
            pl.BlockSpec((1, F1), lambda j: (0, 0)),
        ],
        out_specs=pl.BlockSpec((G, DOUT), lambda j: (0, 0)),
        out_shape=jax.ShapeDtypeStruct((G, DOUT), jnp.float32),
        scratch_shapes=[pltpu.VMEM((G, H), jnp.float32)],
    )(h2, gids3, Wf1, bf1, Wf2, bf2, g4, be4)


def kernel(x, edge_index, graph_ids, params):
    src4 = edge_index[0].reshape(NT, NCHUNK // TB, TB, K)
    dst4 = edge_index[1].reshape(NT, NCHUNK // TB, TB, K)
    src5 = edge_index[0].reshape(2, NT, NCHUNK // (2 * TB), TB, K)
    dst5 = edge_index[1].reshape(2, NT, NCHUNK // (2 * TB), TB, K)
    gids3 = graph_ids.reshape(NB, 1, BN)

    h2 = None
    for i in (1, 2, 3):
        p = params['gc%d' % i]
        bn = params['bn%d' % i]
        if i == 1:
            a2 = _sc_aggregate_edges(x, jnp.zeros_like(x), src5, dst5)
            W1s = p['W1'].reshape(1, -1, H)
        else:
            a2 = _sc_aggregate(h2, src4, dst4)
            W1s = p['W1'].reshape(2, H // 2, H)
        h2 = _tc_layer(
            a2,
            W1s,
            p['b1'].reshape(1, H),
            p['W2'],
            p['b2'].reshape(1, H),
            bn['gamma'].reshape(1, H),
            bn['beta'].reshape(1, H),
            sum_parts=(i == 1),
        )
    return _readout(
        h2, gids3,
        params['fc1']['W'], params['fc1']['b'].reshape(1, -1),
        params['fc2']['W'], params['fc2']['b'].reshape(1, -1),
        params['bn4']['gamma'].reshape(1, -1), params['bn4']['beta'].reshape(1, -1),
    )
